# Initial kernel scaffold; baseline (speedup 1.0000x reference)
import jax
import jax.numpy as jnp
from jax.experimental import pallas as pl


def kernel(x, edge_index_rel0, edge_index_rel1, edge_index_rel2, edge_index_rel3, W_rel0, W_rel1, W_rel2, W_rel3, W_loop, b_loop):
    raise NotImplementedError("write your pallas kernel here")


# speedup vs baseline: 1.7637x; 1.7637x over previous
import functools

import jax
import jax.numpy as jnp
from jax import lax
from jax.experimental import pallas as pl
from jax.experimental.pallas import tpu as pltpu
from jax.experimental.pallas import tpu_sc as plsc

N = 50000
E = 150000
D = 128
R = 4
CHW = 16
NCH = D // CHW
NT = 16
NSC = 2
NP = 50048
RPT = NP // NT
ZR = RPT // 4
GE = 128
G = 78
GB = 13
NB = G // GB
EPT = G * GE
EPAD = NT * EPT
DUMMY = N + 8


def _sc_aggregate(xch, sidx, didx, zeros_in, ones_in):
    mesh = plsc.VectorSubcoreMesh(core_axis_name="c", subcore_axis_name="s")

    @functools.partial(
        pl.kernel,
        out_type=(
            jax.ShapeDtypeStruct((R, NP, D), jnp.float32),
            jax.ShapeDtypeStruct((R, NP, CHW), jnp.float32),
        ),
        mesh=mesh,
        scratch_types=[
            pltpu.VMEM_SHARED((NP, CHW), jnp.float32),
            pltpu.VMEM((G, GE), jnp.int32),
            pltpu.VMEM((G, GE), jnp.int32),
            pltpu.VMEM((GE, CHW), jnp.float32),
            pltpu.VMEM((GE, CHW), jnp.float32),
            pltpu.VMEM((ZR, CHW), jnp.float32),
            pltpu.SemaphoreType.DMA,
        ],
        compiler_params=pltpu.CompilerParams(use_tc_tiling_on_sc=False),
    )
    def sc_kernel(xch_hbm, sidx_hbm, didx_hbm, zeros_hbm, ones_hbm,
                  agg_hbm, deg_hbm,
                  acc_sh, sidx_vm, didx_vm, rows_vm, ones_vm, zeros_vm, sem):
        core = lax.axis_index("c")
        sid = lax.axis_index("s")
        row0 = sid * RPT
        pltpu.sync_copy(zeros_hbm, zeros_vm)
        pltpu.sync_copy(ones_hbm, ones_vm)
        for rr in range(R // NSC):
            r = core * (R // NSC) + rr
            pltpu.sync_copy(didx_hbm.at[r, sid], didx_vm)
            for c in range(NCH + 1):
                for z in range(RPT // ZR):
                    pltpu.sync_copy(zeros_vm,
                                    acc_sh.at[pl.ds(row0 + z * ZR, ZR)])
                if c < NCH:
                    pltpu.sync_copy(sidx_hbm.at[r, c, sid], sidx_vm)
                plsc.subcore_barrier()
                if c < NCH:
                    @pl.loop(0, G)
                    def _blk(g):
                        pltpu.async_copy(
                            xch_hbm.at[sidx_vm.at[g]],
                            rows_vm, sem).wait()
                        pltpu.sync_copy(
                            rows_vm, acc_sh.at[didx_vm.at[g]],
                            add=True)
                else:
                    @pl.loop(0, G)
                    def _deg(g):
                        pltpu.sync_copy(ones_vm, acc_sh.at[didx_vm.at[g]],
                                        add=True)
                plsc.subcore_barrier()
                if c < NCH:
                    pltpu.sync_copy(
                        acc_sh.at[pl.ds(row0, RPT)],
                        agg_hbm.at[r, pl.ds(row0, RPT), pl.ds(c * CHW, CHW)])
                else:
                    pltpu.sync_copy(acc_sh.at[pl.ds(row0, RPT)],
                                    deg_hbm.at[r, pl.ds(row0, RPT)])

    return sc_kernel(xch, sidx, didx, zeros_in, ones_in)


BN = 2000


def _tc_body(agg_ref, deg_ref, x_ref, ws_ref, wl_ref, b_ref, o_ref):
    acc = jnp.dot(x_ref[...], wl_ref[...], preferred_element_type=jnp.float32)
    for r in range(R):
        h = agg_ref[r]
        d = jnp.maximum(deg_ref[r, :, 0:1], 1.0)
        acc = acc + jnp.dot(h / d, ws_ref[r],
                            preferred_element_type=jnp.float32)
    o_ref[...] = acc + b_ref[...]


def _tc_combine(agg, deg, x, ws, wl, b):
    grid = N // BN
    return pl.pallas_call(
        _tc_body,
        grid=(grid,),
        in_specs=[
            pl.BlockSpec((R, BN, D), lambda i: (0, i, 0)),
            pl.BlockSpec((R, BN, CHW), lambda i: (0, i, 0)),
            pl.BlockSpec((BN, D), lambda i: (i, 0)),
            pl.BlockSpec((R, D, D), lambda i: (0, 0, 0)),
            pl.BlockSpec((D, D), lambda i: (0, 0)),
            pl.BlockSpec((1, D), lambda i: (0, 0)),
        ],
        out_specs=pl.BlockSpec((BN, D), lambda i: (i, 0)),
        out_shape=jax.ShapeDtypeStruct((N, D), jnp.float32),
    )(agg, deg, x, ws, wl, b)


def kernel(x, edge_index_rel0, edge_index_rel1, edge_index_rel2,
           edge_index_rel3, W_rel0, W_rel1, W_rel2, W_rel3, W_loop, b_loop):
    x = x.astype(jnp.float32)
    xp = jnp.pad(x, ((0, NP - N), (0, 0)))
    xch = xp.reshape(NP, NCH, CHW).transpose(1, 0, 2).reshape(NCH * NP, CHW)

    srcs, dsts = [], []
    for ei in (edge_index_rel0, edge_index_rel1, edge_index_rel2,
               edge_index_rel3):
        ei = ei.astype(jnp.int32)
        srcs.append(jnp.pad(ei[0], (0, EPAD - E)))
        dsts.append(jnp.pad(ei[1], (0, EPAD - E), constant_values=DUMMY))
    src_all = jnp.stack(srcs)
    dst_all = jnp.stack(dsts)
    offs = jnp.arange(NCH, dtype=jnp.int32) * NP
    sidx = (src_all[:, None, :] + offs[None, :, None]).reshape(
        R, NCH, NT, G, GE)
    didx = dst_all.reshape(R, NT, G, GE)
    zeros_in = jnp.zeros((ZR, CHW), jnp.float32)
    ones_in = jnp.ones((GE, CHW), jnp.float32)

    agg, deg = _sc_aggregate(xch, sidx, didx, zeros_in, ones_in)

    ws = jnp.stack([W_rel0, W_rel1, W_rel2, W_rel3]).astype(jnp.float32)
    return _tc_combine(agg, deg, x, ws, W_loop.astype(jnp.float32),
                       b_loop.astype(jnp.float32).reshape(1, D))


# speedup vs baseline: 1.8543x; 1.0514x over previous
import functools

import jax
import jax.numpy as jnp
from jax import lax
from jax.experimental import pallas as pl
from jax.experimental.pallas import tpu as pltpu
from jax.experimental.pallas import tpu_sc as plsc

N = 50000
E = 150000
D = 128
R = 4
CHW = 16
NCH = D // CHW
NT = 16
NSC = 2
NP = 50048
RPT = NP // NT
ZR = RPT // 4
GE = 128
G = 78
EPT = G * GE
EPAD = NT * EPT
DUMMY = N + 8


def _sc_aggregate(xch, sidx, didx, zeros_in, ones_in):
    mesh = plsc.VectorSubcoreMesh(core_axis_name="c", subcore_axis_name="s")

    @functools.partial(
        pl.kernel,
        out_type=(
            jax.ShapeDtypeStruct((R, NP, D), jnp.float32),
            jax.ShapeDtypeStruct((R, NP, CHW), jnp.float32),
        ),
        mesh=mesh,
        scratch_types=[
            pltpu.VMEM_SHARED((NP, CHW), jnp.float32),
            pltpu.VMEM((G, GE), jnp.int32),
            pltpu.VMEM((G, GE), jnp.int32),
            pltpu.VMEM((4, GE, CHW), jnp.float32),
            pltpu.VMEM((GE, CHW), jnp.float32),
            pltpu.VMEM((ZR, CHW), jnp.float32),
            [pltpu.SemaphoreType.DMA] * 4,
            [pltpu.SemaphoreType.DMA] * 4,
            pltpu.SemaphoreType.DMA,
        ],
        compiler_params=pltpu.CompilerParams(use_tc_tiling_on_sc=False),
    )
    def sc_kernel(xch_hbm, sidx_hbm, didx_hbm, zeros_hbm, ones_hbm,
                  agg_hbm, deg_hbm,
                  acc_sh, sidx_vm, didx_vm, rows_vm, ones_vm, zeros_vm,
                  gsems, ssems, sem):
        core = lax.axis_index("c")
        sid = lax.axis_index("s")
        row0 = sid * RPT
        pltpu.sync_copy(zeros_hbm, zeros_vm)
        pltpu.sync_copy(ones_hbm, ones_vm)
        for rr in range(R // NSC):
            r = core * (R // NSC) + rr
            pltpu.sync_copy(didx_hbm.at[r, sid], didx_vm)
            for c in range(NCH + 1):
                for z in range(RPT // ZR):
                    pltpu.sync_copy(zeros_vm,
                                    acc_sh.at[pl.ds(row0 + z * ZR, ZR)])
                if c < NCH:
                    pltpu.sync_copy(sidx_hbm.at[r, c, sid], sidx_vm)
                plsc.subcore_barrier()
                if c < NCH:
                    def gather(g, u):
                        return pltpu.async_copy(
                            xch_hbm.at[sidx_vm.at[g]],
                            rows_vm.at[u], gsems[u])

                    def scatter(g, u):
                        pltpu.sync_copy(
                            rows_vm.at[u], acc_sh.at[didx_vm.at[g]],
                            add=True)

                    @pl.loop(0, G - G % 4, step=4)
                    def _blk(g):
                        gd = [gather(g + u, u) for u in range(4)]
                        for u in range(4):
                            gd[u].wait()
                            scatter(g + u, u)

                    gd = [gather(G - G % 4 + u, u) for u in range(G % 4)]
                    for u in range(G % 4):
                        gd[u].wait()
                        scatter(G - G % 4 + u, u)
                else:
                    @pl.loop(0, G)
                    def _deg(g):
                        pltpu.sync_copy(ones_vm, acc_sh.at[didx_vm.at[g]],
                                        add=True)
                plsc.subcore_barrier()
                if c < NCH:
                    pltpu.sync_copy(
                        acc_sh.at[pl.ds(row0, RPT)],
                        agg_hbm.at[r, pl.ds(row0, RPT), pl.ds(c * CHW, CHW)])
                else:
                    pltpu.sync_copy(acc_sh.at[pl.ds(row0, RPT)],
                                    deg_hbm.at[r, pl.ds(row0, RPT)])

    return sc_kernel(xch, sidx, didx, zeros_in, ones_in)


BN = 2000


def _tc_body(agg_ref, deg_ref, x_ref, ws_ref, wl_ref, b_ref, o_ref):
    acc = jnp.dot(x_ref[...], wl_ref[...], preferred_element_type=jnp.float32)
    for r in range(R):
        h = agg_ref[r]
        d = jnp.maximum(deg_ref[r, :, 0:1], 1.0)
        acc = acc + jnp.dot(h / d, ws_ref[r],
                            preferred_element_type=jnp.float32)
    o_ref[...] = acc + b_ref[...]


def _tc_combine(agg, deg, x, ws, wl, b):
    grid = N // BN
    return pl.pallas_call(
        _tc_body,
        grid=(grid,),
        in_specs=[
            pl.BlockSpec((R, BN, D), lambda i: (0, i, 0)),
            pl.BlockSpec((R, BN, CHW), lambda i: (0, i, 0)),
            pl.BlockSpec((BN, D), lambda i: (i, 0)),
            pl.BlockSpec((R, D, D), lambda i: (0, 0, 0)),
            pl.BlockSpec((D, D), lambda i: (0, 0)),
            pl.BlockSpec((1, D), lambda i: (0, 0)),
        ],
        out_specs=pl.BlockSpec((BN, D), lambda i: (i, 0)),
        out_shape=jax.ShapeDtypeStruct((N, D), jnp.float32),
    )(agg, deg, x, ws, wl, b)


def kernel(x, edge_index_rel0, edge_index_rel1, edge_index_rel2,
           edge_index_rel3, W_rel0, W_rel1, W_rel2, W_rel3, W_loop, b_loop):
    x = x.astype(jnp.float32)
    xp = jnp.pad(x, ((0, NP - N), (0, 0)))
    xch = xp.reshape(NP, NCH, CHW).transpose(1, 0, 2).reshape(NCH * NP, CHW)

    srcs, dsts = [], []
    for ei in (edge_index_rel0, edge_index_rel1, edge_index_rel2,
               edge_index_rel3):
        ei = ei.astype(jnp.int32)
        srcs.append(jnp.pad(ei[0], (0, EPAD - E)))
        dsts.append(jnp.pad(ei[1], (0, EPAD - E), constant_values=DUMMY))
    src_all = jnp.stack(srcs)
    dst_all = jnp.stack(dsts)
    offs = jnp.arange(NCH, dtype=jnp.int32) * NP
    sidx = (src_all[:, None, :] + offs[None, :, None]).reshape(
        R, NCH, NT, G, GE)
    didx = dst_all.reshape(R, NT, G, GE)
    zeros_in = jnp.zeros((ZR, CHW), jnp.float32)
    ones_in = jnp.ones((GE, CHW), jnp.float32)

    agg, deg = _sc_aggregate(xch, sidx, didx, zeros_in, ones_in)

    ws = jnp.stack([W_rel0, W_rel1, W_rel2, W_rel3]).astype(jnp.float32)
    return _tc_combine(agg, deg, x, ws, W_loop.astype(jnp.float32),
                       b_loop.astype(jnp.float32).reshape(1, D))


# speedup vs baseline: 1.9452x; 1.0490x over previous
import functools

import jax
import jax.numpy as jnp
from jax import lax
from jax.experimental import pallas as pl
from jax.experimental.pallas import tpu as pltpu
from jax.experimental.pallas import tpu_sc as plsc

N = 50000
E = 150000
D = 128
R = 4
CHW = 16
NCH = D // CHW
NT = 16
NSC = 2
NP = 50048
RPT = NP // NT
ZR = RPT // 8
BB = 624
NBLK = 16
EPT = BB * NBLK
EPAD = NT * EPT
DUMMY = N + 8


def _sc_aggregate(xch, sidx, didx, zeros_in, ones_in):
    mesh = plsc.VectorSubcoreMesh(core_axis_name="c", subcore_axis_name="s")

    @functools.partial(
        pl.kernel,
        out_type=(
            jax.ShapeDtypeStruct((R, NP, D), jnp.float32),
            jax.ShapeDtypeStruct((R, NP, CHW), jnp.float32),
        ),
        mesh=mesh,
        scratch_types=[
            pltpu.VMEM_SHARED((NP, CHW), jnp.float32),
            pltpu.VMEM((EPT,), jnp.int32),
            pltpu.VMEM((EPT,), jnp.int32),
            pltpu.VMEM((4, BB, CHW), jnp.float32),
            pltpu.VMEM((BB, CHW), jnp.float32),
            pltpu.VMEM((ZR, CHW), jnp.float32),
            [pltpu.SemaphoreType.DMA] * 4,
            [pltpu.SemaphoreType.DMA] * 4,
            pltpu.SemaphoreType.DMA,
        ],
        compiler_params=pltpu.CompilerParams(use_tc_tiling_on_sc=False),
    )
    def sc_kernel(xch_hbm, sidx_hbm, didx_hbm, zeros_hbm, ones_hbm,
                  agg_hbm, deg_hbm,
                  acc_sh, sidx_vm, didx_vm, rows_vm, ones_vm, zeros_vm,
                  gsems, ssems, sem):
        core = lax.axis_index("c")
        sid = lax.axis_index("s")
        row0 = sid * RPT
        pltpu.sync_copy(zeros_hbm, zeros_vm)
        pltpu.sync_copy(ones_hbm, ones_vm)
        for rr in range(R // NSC):
            r = core * (R // NSC) + rr
            pltpu.sync_copy(didx_hbm.at[r, sid], didx_vm)
            for c in range(NCH + 1):
                for z in range(RPT // ZR):
                    pltpu.sync_copy(zeros_vm,
                                    acc_sh.at[pl.ds(row0 + z * ZR, ZR)])
                if c < NCH:
                    pltpu.sync_copy(sidx_hbm.at[r, c, sid], sidx_vm)
                plsc.subcore_barrier()
                if c < NCH:
                    def gather(g, u):
                        return pltpu.async_copy(
                            xch_hbm.at[sidx_vm.at[pl.ds(g * BB, BB)]],
                            rows_vm.at[u], gsems[u])

                    def scatter(g, u):
                        pltpu.sync_copy(
                            rows_vm.at[u],
                            acc_sh.at[didx_vm.at[pl.ds(g * BB, BB)]],
                            add=True)

                    @pl.loop(0, NBLK, step=4)
                    def _blk(g):
                        gd = [gather(g + u, u) for u in range(4)]
                        for u in range(4):
                            gd[u].wait()
                            scatter(g + u, u)
                else:
                    @pl.loop(0, NBLK)
                    def _deg(g):
                        pltpu.sync_copy(
                            ones_vm,
                            acc_sh.at[didx_vm.at[pl.ds(g * BB, BB)]],
                            add=True)
                plsc.subcore_barrier()
                if c < NCH:
                    pltpu.sync_copy(
                        acc_sh.at[pl.ds(row0, RPT)],
                        agg_hbm.at[r, pl.ds(row0, RPT), pl.ds(c * CHW, CHW)])
                else:
                    pltpu.sync_copy(acc_sh.at[pl.ds(row0, RPT)],
                                    deg_hbm.at[r, pl.ds(row0, RPT)])

    return sc_kernel(xch, sidx, didx, zeros_in, ones_in)


BN = 2000


def _tc_body(agg_ref, deg_ref, x_ref, ws_ref, wl_ref, b_ref, o_ref):
    acc = jnp.dot(x_ref[...], wl_ref[...], preferred_element_type=jnp.float32)
    for r in range(R):
        h = agg_ref[r]
        d = jnp.maximum(deg_ref[r, :, 0:1], 1.0)
        acc = acc + jnp.dot(h / d, ws_ref[r],
                            preferred_element_type=jnp.float32)
    o_ref[...] = acc + b_ref[...]


def _tc_combine(agg, deg, x, ws, wl, b):
    grid = N // BN
    return pl.pallas_call(
        _tc_body,
        grid=(grid,),
        in_specs=[
            pl.BlockSpec((R, BN, D), lambda i: (0, i, 0)),
            pl.BlockSpec((R, BN, CHW), lambda i: (0, i, 0)),
            pl.BlockSpec((BN, D), lambda i: (i, 0)),
            pl.BlockSpec((R, D, D), lambda i: (0, 0, 0)),
            pl.BlockSpec((D, D), lambda i: (0, 0)),
            pl.BlockSpec((1, D), lambda i: (0, 0)),
        ],
        out_specs=pl.BlockSpec((BN, D), lambda i: (i, 0)),
        out_shape=jax.ShapeDtypeStruct((N, D), jnp.float32),
    )(agg, deg, x, ws, wl, b)


def kernel(x, edge_index_rel0, edge_index_rel1, edge_index_rel2,
           edge_index_rel3, W_rel0, W_rel1, W_rel2, W_rel3, W_loop, b_loop):
    x = x.astype(jnp.float32)
    xp = jnp.pad(x, ((0, NP - N), (0, 0)))
    xch = xp.reshape(NP, NCH, CHW).transpose(1, 0, 2).reshape(NCH * NP, CHW)

    srcs, dsts = [], []
    for ei in (edge_index_rel0, edge_index_rel1, edge_index_rel2,
               edge_index_rel3):
        ei = ei.astype(jnp.int32)
        srcs.append(jnp.pad(ei[0], (0, EPAD - E)))
        dsts.append(jnp.pad(ei[1], (0, EPAD - E), constant_values=DUMMY))
    src_all = jnp.stack(srcs)
    dst_all = jnp.stack(dsts)
    offs = jnp.arange(NCH, dtype=jnp.int32) * NP
    sidx = (src_all[:, None, :] + offs[None, :, None]).reshape(
        R, NCH, NT, EPT)
    didx = dst_all.reshape(R, NT, EPT)
    zeros_in = jnp.zeros((ZR, CHW), jnp.float32)
    ones_in = jnp.ones((BB, CHW), jnp.float32)

    agg, deg = _sc_aggregate(xch, sidx, didx, zeros_in, ones_in)

    ws = jnp.stack([W_rel0, W_rel1, W_rel2, W_rel3]).astype(jnp.float32)
    return _tc_combine(agg, deg, x, ws, W_loop.astype(jnp.float32),
                       b_loop.astype(jnp.float32).reshape(1, D))


# speedup vs baseline: 2.0783x; 1.0684x over previous
import functools

import jax
import jax.numpy as jnp
from jax import lax
from jax.experimental import pallas as pl
from jax.experimental.pallas import tpu as pltpu
from jax.experimental.pallas import tpu_sc as plsc

N = 50000
E = 150000
D = 128
R = 4
CHW = 32
NCH = D // CHW
NT = 16
NSC = 2
NP = 50048
RPT = NP // NT
ZR = 184
BB = 208
NBUF = 3
QE = 2496
NQ = 4
EPT = QE * NQ
EPAD = NT * EPT
DUMMY = N + 8


def _sc_aggregate(xch, sidx, didx, zeros_in):
    mesh = plsc.VectorSubcoreMesh(core_axis_name="c", subcore_axis_name="s")

    @functools.partial(
        pl.kernel,
        out_type=(
            jax.ShapeDtypeStruct((R, NP, D), jnp.float32),
            jax.ShapeDtypeStruct((R, NP, CHW), jnp.float32),
        ),
        mesh=mesh,
        scratch_types=[
            pltpu.VMEM_SHARED((NP, CHW), jnp.float32),
            pltpu.VMEM((QE,), jnp.int32),
            pltpu.VMEM((QE,), jnp.int32),
            pltpu.VMEM((NBUF, BB, CHW), jnp.float32),
            pltpu.VMEM((ZR, CHW), jnp.float32),
            [pltpu.SemaphoreType.DMA] * NBUF,
            pltpu.SemaphoreType.DMA,
        ],
        compiler_params=pltpu.CompilerParams(use_tc_tiling_on_sc=False),
    )
    def sc_kernel(xch_hbm, sidx_hbm, didx_hbm, zeros_hbm,
                  agg_hbm, deg_hbm,
                  acc_sh, sidx_vm, didx_vm, rows_vm, zeros_vm,
                  gsems, sem):
        core = lax.axis_index("c")
        sid = lax.axis_index("s")
        row0 = sid * RPT
        pltpu.sync_copy(zeros_hbm, zeros_vm)

        def zero_acc():
            for z in range(RPT // ZR):
                pltpu.sync_copy(zeros_vm,
                                acc_sh.at[pl.ds(row0 + z * ZR, ZR)])

        def gather(g, u):
            return pltpu.async_copy(
                xch_hbm.at[sidx_vm.at[pl.ds(g * BB, BB)]],
                rows_vm.at[u], gsems[u])

        def scatter(g, u):
            pltpu.sync_copy(rows_vm.at[u],
                            acc_sh.at[didx_vm.at[pl.ds(g * BB, BB)]],
                            add=True)

        for rr in range(R // NSC):
            r = core * (R // NSC) + rr
            for c in range(NCH):
                zero_acc()
                plsc.subcore_barrier()

                @pl.loop(0, NQ)
                def _quarter(q):
                    pltpu.sync_copy(
                        sidx_hbm.at[r, c, sid, pl.ds(q * QE, QE)], sidx_vm)
                    pltpu.sync_copy(
                        didx_hbm.at[r, sid, pl.ds(q * QE, QE)], didx_vm)

                    @pl.loop(0, QE // BB, step=NBUF)
                    def _blk(g):
                        gd = [gather(g + u, u) for u in range(NBUF)]
                        for u in range(NBUF):
                            gd[u].wait()
                            scatter(g + u, u)

                plsc.subcore_barrier()
                pltpu.sync_copy(
                    acc_sh.at[pl.ds(row0, RPT)],
                    agg_hbm.at[r, pl.ds(row0, RPT), pl.ds(c * CHW, CHW)])

            zero_acc()

            @pl.loop(0, BB)
            def _ones(i):
                rows_vm[0, i, pl.ds(0, 16)] = jnp.full((16,), 1.0,
                                                       jnp.float32)
                rows_vm[0, i, pl.ds(16, 16)] = jnp.full((16,), 1.0,
                                                        jnp.float32)

            plsc.subcore_barrier()

            @pl.loop(0, NQ)
            def _dquarter(q):
                pltpu.sync_copy(
                    didx_hbm.at[r, sid, pl.ds(q * QE, QE)], didx_vm)

                @pl.loop(0, QE // BB)
                def _dblk(g):
                    scatter(g, 0)

            plsc.subcore_barrier()
            pltpu.sync_copy(acc_sh.at[pl.ds(row0, RPT)],
                            deg_hbm.at[r, pl.ds(row0, RPT)])

    return sc_kernel(xch, sidx, didx, zeros_in)


BN = 2000


def _tc_body(agg_ref, deg_ref, x_ref, ws_ref, wl_ref, b_ref, o_ref):
    acc = jnp.dot(x_ref[...], wl_ref[...], preferred_element_type=jnp.float32)
    for r in range(R):
        h = agg_ref[r]
        d = jnp.maximum(deg_ref[r, :, 0:1], 1.0)
        acc = acc + jnp.dot(h / d, ws_ref[r],
                            preferred_element_type=jnp.float32)
    o_ref[...] = acc + b_ref[...]


def _tc_combine(agg, deg, x, ws, wl, b):
    grid = N // BN
    return pl.pallas_call(
        _tc_body,
        grid=(grid,),
        in_specs=[
            pl.BlockSpec((R, BN, D), lambda i: (0, i, 0)),
            pl.BlockSpec((R, BN, CHW), lambda i: (0, i, 0)),
            pl.BlockSpec((BN, D), lambda i: (i, 0)),
            pl.BlockSpec((R, D, D), lambda i: (0, 0, 0)),
            pl.BlockSpec((D, D), lambda i: (0, 0)),
            pl.BlockSpec((1, D), lambda i: (0, 0)),
        ],
        out_specs=pl.BlockSpec((BN, D), lambda i: (i, 0)),
        out_shape=jax.ShapeDtypeStruct((N, D), jnp.float32),
    )(agg, deg, x, ws, wl, b)


def kernel(x, edge_index_rel0, edge_index_rel1, edge_index_rel2,
           edge_index_rel3, W_rel0, W_rel1, W_rel2, W_rel3, W_loop, b_loop):
    x = x.astype(jnp.float32)
    xp = jnp.pad(x, ((0, NP - N), (0, 0)))
    xch = xp.reshape(NP, NCH, CHW).transpose(1, 0, 2).reshape(NCH * NP, CHW)

    srcs, dsts = [], []
    for ei in (edge_index_rel0, edge_index_rel1, edge_index_rel2,
               edge_index_rel3):
        ei = ei.astype(jnp.int32)
        srcs.append(jnp.pad(ei[0], (0, EPAD - E)))
        dsts.append(jnp.pad(ei[1], (0, EPAD - E), constant_values=DUMMY))
    src_all = jnp.stack(srcs)
    dst_all = jnp.stack(dsts)
    offs = jnp.arange(NCH, dtype=jnp.int32) * NP
    sidx = (src_all[:, None, :] + offs[None, :, None]).reshape(
        R, NCH, NT, EPT)
    didx = dst_all.reshape(R, NT, EPT)
    zeros_in = jnp.zeros((ZR, CHW), jnp.float32)

    agg, deg = _sc_aggregate(xch, sidx, didx, zeros_in)

    ws = jnp.stack([W_rel0, W_rel1, W_rel2, W_rel3]).astype(jnp.float32)
    return _tc_combine(agg, deg, x, ws, W_loop.astype(jnp.float32),
                       b_loop.astype(jnp.float32).reshape(1, D))


# speedup vs baseline: 2.9073x; 1.3989x over previous
import functools

import jax
import jax.numpy as jnp
from jax import lax
from jax.experimental import pallas as pl
from jax.experimental.pallas import tpu as pltpu
from jax.experimental.pallas import tpu_sc as plsc

N = 50000
E = 150000
D = 128
R = 4
CHW = 64
NCH = D // CHW
NT = 16
NSC = 2
NP = 50048
RPT = NP // NT
ZR = 184
BB = 208
NBUF = 3
QE = 2496
NQ = 4
EPT = QE * NQ
EPAD = NT * EPT
DUMMY = N + 8


def _sc_aggregate(xch, sidx, didx, zeros_in):
    mesh = plsc.VectorSubcoreMesh(core_axis_name="c", subcore_axis_name="s")

    @functools.partial(
        pl.kernel,
        out_type=(
            jax.ShapeDtypeStruct((R, NP, D), jnp.bfloat16),
            jax.ShapeDtypeStruct((R, NP, CHW), jnp.bfloat16),
        ),
        mesh=mesh,
        scratch_types=[
            pltpu.VMEM_SHARED((NP, CHW), jnp.bfloat16),
            pltpu.VMEM((QE,), jnp.int32),
            pltpu.VMEM((QE,), jnp.int32),
            pltpu.VMEM((NBUF, BB, CHW), jnp.bfloat16),
            pltpu.VMEM((ZR, CHW), jnp.bfloat16),
            [pltpu.SemaphoreType.DMA] * NBUF,
            pltpu.SemaphoreType.DMA,
        ],
        compiler_params=pltpu.CompilerParams(use_tc_tiling_on_sc=False),
    )
    def sc_kernel(xch_hbm, sidx_hbm, didx_hbm, zeros_hbm,
                  agg_hbm, deg_hbm,
                  acc_sh, sidx_vm, didx_vm, rows_vm, zeros_vm,
                  gsems, sem):
        core = lax.axis_index("c")
        sid = lax.axis_index("s")
        row0 = sid * RPT
        pltpu.sync_copy(zeros_hbm, zeros_vm)

        def zero_acc():
            for z in range(RPT // ZR):
                pltpu.sync_copy(zeros_vm,
                                acc_sh.at[pl.ds(row0 + z * ZR, ZR)])

        def gather(g, u):
            return pltpu.async_copy(
                xch_hbm.at[sidx_vm.at[pl.ds(g * BB, BB)]],
                rows_vm.at[u], gsems[u])

        def scatter(g, u):
            pltpu.sync_copy(rows_vm.at[u],
                            acc_sh.at[didx_vm.at[pl.ds(g * BB, BB)]],
                            add=True)

        for rr in range(R // NSC):
            r = core * (R // NSC) + rr
            for c in range(NCH):
                zero_acc()
                plsc.subcore_barrier()

                @pl.loop(0, NQ)
                def _quarter(q):
                    pltpu.sync_copy(
                        sidx_hbm.at[r, c, sid, pl.ds(q * QE, QE)], sidx_vm)
                    pltpu.sync_copy(
                        didx_hbm.at[r, sid, pl.ds(q * QE, QE)], didx_vm)

                    @pl.loop(0, QE // BB, step=NBUF)
                    def _blk(g):
                        gd = [gather(g + u, u) for u in range(NBUF)]
                        for u in range(NBUF):
                            gd[u].wait()
                            scatter(g + u, u)

                plsc.subcore_barrier()
                pltpu.sync_copy(
                    acc_sh.at[pl.ds(row0, RPT)],
                    agg_hbm.at[r, pl.ds(row0, RPT), pl.ds(c * CHW, CHW)])

            zero_acc()

            @pl.loop(0, BB)
            def _ones(i):
                rows_vm[0, i, pl.ds(0, 32)] = jnp.full((32,), 1.0,
                                                       jnp.bfloat16)
                rows_vm[0, i, pl.ds(32, 32)] = jnp.full((32,), 1.0,
                                                        jnp.bfloat16)

            plsc.subcore_barrier()

            @pl.loop(0, NQ)
            def _dquarter(q):
                pltpu.sync_copy(
                    didx_hbm.at[r, sid, pl.ds(q * QE, QE)], didx_vm)

                @pl.loop(0, QE // BB)
                def _dblk(g):
                    scatter(g, 0)

            plsc.subcore_barrier()
            pltpu.sync_copy(acc_sh.at[pl.ds(row0, RPT)],
                            deg_hbm.at[r, pl.ds(row0, RPT)])

    return sc_kernel(xch, sidx, didx, zeros_in)


BN = 2000


def _tc_body(agg_ref, deg_ref, x_ref, ws_ref, wl_ref, b_ref, o_ref):
    acc = jnp.dot(x_ref[...], wl_ref[...], preferred_element_type=jnp.float32)
    for r in range(R):
        h = agg_ref[r].astype(jnp.float32)
        d = jnp.maximum(deg_ref[r, :, 0:1].astype(jnp.float32), 1.0)
        acc = acc + jnp.dot(h / d, ws_ref[r],
                            preferred_element_type=jnp.float32)
    o_ref[...] = acc + b_ref[...]


def _tc_combine(agg, deg, x, ws, wl, b):
    grid = N // BN
    return pl.pallas_call(
        _tc_body,
        grid=(grid,),
        in_specs=[
            pl.BlockSpec((R, BN, D), lambda i: (0, i, 0)),
            pl.BlockSpec((R, BN, CHW), lambda i: (0, i, 0)),
            pl.BlockSpec((BN, D), lambda i: (i, 0)),
            pl.BlockSpec((R, D, D), lambda i: (0, 0, 0)),
            pl.BlockSpec((D, D), lambda i: (0, 0)),
            pl.BlockSpec((1, D), lambda i: (0, 0)),
        ],
        out_specs=pl.BlockSpec((BN, D), lambda i: (i, 0)),
        out_shape=jax.ShapeDtypeStruct((N, D), jnp.float32),
    )(agg, deg, x, ws, wl, b)


def kernel(x, edge_index_rel0, edge_index_rel1, edge_index_rel2,
           edge_index_rel3, W_rel0, W_rel1, W_rel2, W_rel3, W_loop, b_loop):
    x = x.astype(jnp.float32)
    xp = jnp.pad(x, ((0, NP - N), (0, 0))).astype(jnp.bfloat16)
    xch = xp.reshape(NP, NCH, CHW).transpose(1, 0, 2).reshape(NCH * NP, CHW)

    srcs, dsts = [], []
    for ei in (edge_index_rel0, edge_index_rel1, edge_index_rel2,
               edge_index_rel3):
        ei = ei.astype(jnp.int32)
        srcs.append(jnp.pad(ei[0], (0, EPAD - E)))
        dsts.append(jnp.pad(ei[1], (0, EPAD - E), constant_values=DUMMY))
    src_all = jnp.stack(srcs)
    dst_all = jnp.stack(dsts)
    offs = jnp.arange(NCH, dtype=jnp.int32) * NP
    sidx = (src_all[:, None, :] + offs[None, :, None]).reshape(
        R, NCH, NT, EPT)
    didx = dst_all.reshape(R, NT, EPT)
    zeros_in = jnp.zeros((ZR, CHW), jnp.bfloat16)

    agg, deg = _sc_aggregate(xch, sidx, didx, zeros_in)

    ws = jnp.stack([W_rel0, W_rel1, W_rel2, W_rel3]).astype(jnp.float32)
    return _tc_combine(agg, deg, x, ws, W_loop.astype(jnp.float32),
                       b_loop.astype(jnp.float32).reshape(1, D))


# speedup vs baseline: 2.9255x; 1.0063x over previous
import functools

import jax
import jax.numpy as jnp
from jax import lax
from jax.experimental import pallas as pl
from jax.experimental.pallas import tpu as pltpu
from jax.experimental.pallas import tpu_sc as plsc

N = 50000
E = 150000
D = 128
R = 4
CHW = 64
NCH = D // CHW
NT = 16
NSC = 2
NP = 50048
RPT = NP // NT
ZR = 184
BB = 208
NBUF = 3
QE = 2496
NQ = 4
EPT = QE * NQ
EPAD = NT * EPT
DUMMY = N + 8


def _sc_aggregate(xch, sidx, didx, zeros_in):
    mesh = plsc.VectorSubcoreMesh(core_axis_name="c", subcore_axis_name="s")

    @functools.partial(
        pl.kernel,
        out_type=(
            jax.ShapeDtypeStruct((NSC, NP, D), jnp.bfloat16),
            jax.ShapeDtypeStruct((NSC, NP, CHW), jnp.bfloat16),
        ),
        mesh=mesh,
        scratch_types=[
            pltpu.VMEM_SHARED((NP, CHW), jnp.bfloat16),
            pltpu.VMEM((QE,), jnp.int32),
            pltpu.VMEM((QE,), jnp.int32),
            pltpu.VMEM((NBUF, BB, CHW), jnp.bfloat16),
            pltpu.VMEM((ZR, CHW), jnp.bfloat16),
            [pltpu.SemaphoreType.DMA] * NBUF,
            pltpu.SemaphoreType.DMA,
        ],
        compiler_params=pltpu.CompilerParams(use_tc_tiling_on_sc=False),
    )
    def sc_kernel(xch_hbm, sidx_hbm, didx_hbm, zeros_hbm,
                  agg_hbm, deg_hbm,
                  acc_sh, sidx_vm, didx_vm, rows_vm, zeros_vm,
                  gsems, sem):
        core = lax.axis_index("c")
        sid = lax.axis_index("s")
        row0 = sid * RPT
        pltpu.sync_copy(zeros_hbm, zeros_vm)

        def zero_acc():
            for z in range(RPT // ZR):
                pltpu.sync_copy(zeros_vm,
                                acc_sh.at[pl.ds(row0 + z * ZR, ZR)])

        def gather(g, u):
            return pltpu.async_copy(
                xch_hbm.at[sidx_vm.at[pl.ds(g * BB, BB)]],
                rows_vm.at[u], gsems[u])

        def scatter(g, u):
            pltpu.sync_copy(rows_vm.at[u],
                            acc_sh.at[didx_vm.at[pl.ds(g * BB, BB)]],
                            add=True)

        for c in range(NCH):
            zero_acc()
            plsc.subcore_barrier()

            @pl.loop(0, NQ)
            def _quarter(q):
                pltpu.sync_copy(
                    sidx_hbm.at[core, c, sid, pl.ds(q * QE, QE)], sidx_vm)
                pltpu.sync_copy(
                    didx_hbm.at[core, sid, pl.ds(q * QE, QE)], didx_vm)

                @pl.loop(0, QE // BB, step=NBUF)
                def _blk(g):
                    gd = [gather(g + u, u) for u in range(NBUF)]
                    for u in range(NBUF):
                        gd[u].wait()
                        scatter(g + u, u)

            plsc.subcore_barrier()
            pltpu.sync_copy(
                acc_sh.at[pl.ds(row0, RPT)],
                agg_hbm.at[core, pl.ds(row0, RPT), pl.ds(c * CHW, CHW)])

        zero_acc()

        @pl.loop(0, BB)
        def _ones(i):
            rows_vm[0, i, pl.ds(0, 32)] = jnp.full((32,), 1.0, jnp.bfloat16)
            rows_vm[0, i, pl.ds(32, 32)] = jnp.full((32,), 1.0, jnp.bfloat16)

        plsc.subcore_barrier()

        @pl.loop(0, NQ)
        def _dquarter(q):
            pltpu.sync_copy(
                didx_hbm.at[core, sid, pl.ds(q * QE, QE)], didx_vm)

            @pl.loop(0, QE // BB)
            def _dblk(g):
                scatter(g, 0)

        plsc.subcore_barrier()
        pltpu.sync_copy(acc_sh.at[pl.ds(row0, RPT)],
                        deg_hbm.at[core, pl.ds(row0, RPT)])

    return sc_kernel(xch, sidx, didx, zeros_in)


BN = 2000


def _tc_body(agg_ref, deg_ref, base_ref, ws_ref, o_ref):
    acc = base_ref[...]
    for r in range(NSC):
        h = agg_ref[r].astype(jnp.float32)
        d = jnp.maximum(deg_ref[r, :, 0:1].astype(jnp.float32), 1.0)
        acc = acc + jnp.dot(h / d, ws_ref[r],
                            preferred_element_type=jnp.float32)
    o_ref[...] = acc


def _tc_combine(agg, deg, base, ws):
    grid = N // BN
    return pl.pallas_call(
        _tc_body,
        grid=(grid,),
        in_specs=[
            pl.BlockSpec((NSC, BN, D), lambda i: (0, i, 0)),
            pl.BlockSpec((NSC, BN, CHW), lambda i: (0, i, 0)),
            pl.BlockSpec((BN, D), lambda i: (i, 0)),
            pl.BlockSpec((NSC, D, D), lambda i: (0, 0, 0)),
        ],
        out_specs=pl.BlockSpec((BN, D), lambda i: (i, 0)),
        out_shape=jax.ShapeDtypeStruct((N, D), jnp.float32),
    )(agg, deg, base, ws)


def _tc_selfloop_body(x_ref, wl_ref, b_ref, o_ref):
    o_ref[...] = jnp.dot(x_ref[...], wl_ref[...],
                         preferred_element_type=jnp.float32) + b_ref[...]


def _tc_selfloop(x, wl, b):
    grid = N // BN
    return pl.pallas_call(
        _tc_selfloop_body,
        grid=(grid,),
        in_specs=[
            pl.BlockSpec((BN, D), lambda i: (i, 0)),
            pl.BlockSpec((D, D), lambda i: (0, 0)),
            pl.BlockSpec((1, D), lambda i: (0, 0)),
        ],
        out_specs=pl.BlockSpec((BN, D), lambda i: (i, 0)),
        out_shape=jax.ShapeDtypeStruct((N, D), jnp.float32),
    )(x, wl, b)


def kernel(x, edge_index_rel0, edge_index_rel1, edge_index_rel2,
           edge_index_rel3, W_rel0, W_rel1, W_rel2, W_rel3, W_loop, b_loop):
    x = x.astype(jnp.float32)
    xp = jnp.pad(x, ((0, NP - N), (0, 0))).astype(jnp.bfloat16)
    xch = xp.reshape(NP, NCH, CHW).transpose(1, 0, 2).reshape(NCH * NP, CHW)

    edges = (edge_index_rel0, edge_index_rel2, edge_index_rel1,
             edge_index_rel3)
    srcs, dsts = [], []
    for ei in edges:
        ei = ei.astype(jnp.int32)
        srcs.append(jnp.pad(ei[0], (0, EPAD - E)))
        dsts.append(jnp.pad(ei[1], (0, EPAD - E), constant_values=DUMMY))
    src_all = jnp.stack(srcs)
    dst_all = jnp.stack(dsts)
    offs = jnp.arange(NCH, dtype=jnp.int32) * NP
    sidx = (src_all[:, None, :] + offs[None, :, None]).reshape(
        2, NSC, NCH, NT, EPT)
    didx = dst_all.reshape(2, NSC, NT, EPT)
    zeros_in = jnp.zeros((ZR, CHW), jnp.bfloat16)

    ws02 = jnp.stack([W_rel0, W_rel2]).astype(jnp.float32)
    ws13 = jnp.stack([W_rel1, W_rel3]).astype(jnp.float32)

    agg0, deg0 = _sc_aggregate(xch, sidx[0], didx[0], zeros_in)
    agg1, deg1 = _sc_aggregate(xch, sidx[1], didx[1], zeros_in)

    y = _tc_selfloop(x, W_loop.astype(jnp.float32),
                     b_loop.astype(jnp.float32).reshape(1, D))
    y = _tc_combine(agg0, deg0, y, ws02)
    y = _tc_combine(agg1, deg1, y, ws13)
    return y


# speedup vs baseline: 2.9427x; 1.0059x over previous
import functools

import jax
import jax.numpy as jnp
from jax import lax
from jax.experimental import pallas as pl
from jax.experimental.pallas import tpu as pltpu
from jax.experimental.pallas import tpu_sc as plsc

N = 50000
E = 150000
D = 128
R = 4
CHW = 64
NCH = D // CHW
NT = 16
NSC = 2
NP = 50048
RPT = NP // NT
ZR = 184
BB = 208
NBUF = 3
QE = 2496
NQ = 4
EPT = QE * NQ
EPAD = NT * EPT
DUMMY = N + 8


def _sc_aggregate(xch, sidx, didx, zeros_in):
    mesh = plsc.VectorSubcoreMesh(core_axis_name="c", subcore_axis_name="s")

    @functools.partial(
        pl.kernel,
        out_type=(
            jax.ShapeDtypeStruct((NSC, NP, D), jnp.bfloat16),
            jax.ShapeDtypeStruct((NSC, NP, CHW), jnp.bfloat16),
        ),
        mesh=mesh,
        scratch_types=[
            pltpu.VMEM_SHARED((NP, CHW), jnp.bfloat16),
            pltpu.VMEM((QE,), jnp.int32),
            pltpu.VMEM((QE,), jnp.int32),
            pltpu.VMEM((NBUF, BB, CHW), jnp.bfloat16),
            pltpu.VMEM((ZR, CHW), jnp.bfloat16),
            [pltpu.SemaphoreType.DMA] * NBUF,
            pltpu.SemaphoreType.DMA,
        ],
        compiler_params=pltpu.CompilerParams(use_tc_tiling_on_sc=False),
    )
    def sc_kernel(xch_hbm, sidx_hbm, didx_hbm, zeros_hbm,
                  agg_hbm, deg_hbm,
                  acc_sh, sidx_vm, didx_vm, rows_vm, zeros_vm,
                  gsems, sem):
        core = lax.axis_index("c")
        sid = lax.axis_index("s")
        row0 = sid * RPT
        pltpu.sync_copy(zeros_hbm, zeros_vm)

        def zero_acc():
            for z in range(RPT // ZR):
                pltpu.sync_copy(zeros_vm,
                                acc_sh.at[pl.ds(row0 + z * ZR, ZR)])

        def gather(g, u):
            return pltpu.async_copy(
                xch_hbm.at[sidx_vm.at[pl.ds(g * BB, BB)]],
                rows_vm.at[u], gsems[u])

        def wait_gather(u):
            pltpu.make_async_copy(
                xch_hbm.at[sidx_vm.at[pl.ds(0, BB)]],
                rows_vm.at[u], gsems[u]).wait()

        def scatter(g, u):
            pltpu.sync_copy(rows_vm.at[u],
                            acc_sh.at[didx_vm.at[pl.ds(g * BB, BB)]],
                            add=True)

        for c in range(NCH):
            zero_acc()
            plsc.subcore_barrier()

            @pl.loop(0, NQ)
            def _quarter(q):
                pltpu.sync_copy(
                    sidx_hbm.at[core, c, sid, pl.ds(q * QE, QE)], sidx_vm)
                pltpu.sync_copy(
                    didx_hbm.at[core, sid, pl.ds(q * QE, QE)], didx_vm)

                for u in range(NBUF):
                    gather(u, u)

                @pl.loop(0, QE // BB, step=NBUF)
                def _blk(g):
                    for u in range(NBUF):
                        wait_gather(u)
                        scatter(g + u, u)

                        @pl.when(g + u + NBUF < QE // BB)
                        def _fire():
                            gather(g + u + NBUF, u)

            plsc.subcore_barrier()
            pltpu.sync_copy(
                acc_sh.at[pl.ds(row0, RPT)],
                agg_hbm.at[core, pl.ds(row0, RPT), pl.ds(c * CHW, CHW)])

        zero_acc()

        @pl.loop(0, BB)
        def _ones(i):
            rows_vm[0, i, pl.ds(0, 32)] = jnp.full((32,), 1.0, jnp.bfloat16)
            rows_vm[0, i, pl.ds(32, 32)] = jnp.full((32,), 1.0, jnp.bfloat16)

        plsc.subcore_barrier()

        @pl.loop(0, NQ)
        def _dquarter(q):
            pltpu.sync_copy(
                didx_hbm.at[core, sid, pl.ds(q * QE, QE)], didx_vm)

            @pl.loop(0, QE // BB)
            def _dblk(g):
                scatter(g, 0)

        plsc.subcore_barrier()
        pltpu.sync_copy(acc_sh.at[pl.ds(row0, RPT)],
                        deg_hbm.at[core, pl.ds(row0, RPT)])

    return sc_kernel(xch, sidx, didx, zeros_in)


BN = 2000


def _tc_body(agg_ref, deg_ref, base_ref, ws_ref, o_ref):
    acc = base_ref[...]
    for r in range(NSC):
        h = agg_ref[r].astype(jnp.float32)
        d = jnp.maximum(deg_ref[r, :, 0:1].astype(jnp.float32), 1.0)
        acc = acc + jnp.dot(h / d, ws_ref[r],
                            preferred_element_type=jnp.float32)
    o_ref[...] = acc


def _tc_combine(agg, deg, base, ws):
    grid = N // BN
    return pl.pallas_call(
        _tc_body,
        grid=(grid,),
        in_specs=[
            pl.BlockSpec((NSC, BN, D), lambda i: (0, i, 0)),
            pl.BlockSpec((NSC, BN, CHW), lambda i: (0, i, 0)),
            pl.BlockSpec((BN, D), lambda i: (i, 0)),
            pl.BlockSpec((NSC, D, D), lambda i: (0, 0, 0)),
        ],
        out_specs=pl.BlockSpec((BN, D), lambda i: (i, 0)),
        out_shape=jax.ShapeDtypeStruct((N, D), jnp.float32),
    )(agg, deg, base, ws)


def _tc_selfloop_body(x_ref, wl_ref, b_ref, o_ref):
    o_ref[...] = jnp.dot(x_ref[...], wl_ref[...],
                         preferred_element_type=jnp.float32) + b_ref[...]


def _tc_selfloop(x, wl, b):
    grid = N // BN
    return pl.pallas_call(
        _tc_selfloop_body,
        grid=(grid,),
        in_specs=[
            pl.BlockSpec((BN, D), lambda i: (i, 0)),
            pl.BlockSpec((D, D), lambda i: (0, 0)),
            pl.BlockSpec((1, D), lambda i: (0, 0)),
        ],
        out_specs=pl.BlockSpec((BN, D), lambda i: (i, 0)),
        out_shape=jax.ShapeDtypeStruct((N, D), jnp.float32),
    )(x, wl, b)


def kernel(x, edge_index_rel0, edge_index_rel1, edge_index_rel2,
           edge_index_rel3, W_rel0, W_rel1, W_rel2, W_rel3, W_loop, b_loop):
    x = x.astype(jnp.float32)
    xp = jnp.pad(x, ((0, NP - N), (0, 0))).astype(jnp.bfloat16)
    xch = xp.reshape(NP, NCH, CHW).transpose(1, 0, 2).reshape(NCH * NP, CHW)

    edges = (edge_index_rel0, edge_index_rel2, edge_index_rel1,
             edge_index_rel3)
    srcs, dsts = [], []
    for ei in edges:
        ei = ei.astype(jnp.int32)
        srcs.append(jnp.pad(ei[0], (0, EPAD - E)))
        dsts.append(jnp.pad(ei[1], (0, EPAD - E), constant_values=DUMMY))
    src_all = jnp.stack(srcs)
    dst_all = jnp.stack(dsts)
    offs = jnp.arange(NCH, dtype=jnp.int32) * NP
    sidx = (src_all[:, None, :] + offs[None, :, None]).reshape(
        2, NSC, NCH, NT, EPT)
    didx = dst_all.reshape(2, NSC, NT, EPT)
    zeros_in = jnp.zeros((ZR, CHW), jnp.bfloat16)

    ws02 = jnp.stack([W_rel0, W_rel2]).astype(jnp.float32)
    ws13 = jnp.stack([W_rel1, W_rel3]).astype(jnp.float32)

    agg0, deg0 = _sc_aggregate(xch, sidx[0], didx[0], zeros_in)
    agg1, deg1 = _sc_aggregate(xch, sidx[1], didx[1], zeros_in)

    y = _tc_selfloop(x, W_loop.astype(jnp.float32),
                     b_loop.astype(jnp.float32).reshape(1, D))
    y = _tc_combine(agg0, deg0, y, ws02)
    y = _tc_combine(agg1, deg1, y, ws13)
    return y
